# baseline (speedup 1.0000x reference)
import functools

import jax
import jax.numpy as jnp
from jax import lax
from jax.experimental import pallas as pl
from jax.experimental.pallas import tpu as pltpu
from jax.experimental.pallas import tpu_sc as plsc

N = 10000
E = 320000
D = 128
LANES = 16
CHUNK = 128
NC = 2
NS = 16
NW = NC * NS
ITERS = 81
PADCHUNKS = ITERS * NW
E_PAD = PADCHUNKS * CHUNK
EW = ITERS * 3 * CHUNK
RBLK = 80
NRBLK = N // RBLK
RITERS = -(-NRBLK // NS)


def _mm_body(x_ref, w_ref, b_ref, o_ref):
    o_ref[...] = (
        jnp.dot(x_ref[...], w_ref[...], preferred_element_type=jnp.float32)
        + b_ref[...]
    )


def _matmul_bias(x, W, b):
    M = x.shape[0]
    BM = 1000
    return pl.pallas_call(
        _mm_body,
        grid=(M // BM,),
        in_specs=[
            pl.BlockSpec((BM, D), lambda i: (i, 0)),
            pl.BlockSpec((D, D), lambda i: (0, 0)),
            pl.BlockSpec((1, D), lambda i: (0, 0)),
        ],
        out_specs=pl.BlockSpec((BM, D), lambda i: (i, 0)),
        out_shape=jax.ShapeDtypeStruct((M, D), jnp.float32),
    )(x, W, b.reshape(1, D))


def _add_body(a_ref, b_ref, o_ref):
    o_ref[...] = a_ref[...] + b_ref[...]


def _add2(a, b):
    BM = 1000
    return pl.pallas_call(
        _add_body,
        grid=(N // BM,),
        in_specs=[pl.BlockSpec((BM, D), lambda i: (i, 0))] * 2,
        out_specs=pl.BlockSpec((BM, D), lambda i: (i, 0)),
        out_shape=jax.ShapeDtypeStruct((N, D), jnp.float32),
    )(a, b)


def _sc_scatter(h, edata):
    mesh = plsc.VectorSubcoreMesh(core_axis_name="c", subcore_axis_name="s")

    @functools.partial(
        pl.kernel,
        out_type=jax.ShapeDtypeStruct((NC, N, D), jnp.float32),
        mesh=mesh,
        compiler_params=pltpu.CompilerParams(needs_layout_passes=False),
        scratch_types=(
            pltpu.VMEM((CHUNK, D), jnp.float32),
            pltpu.VMEM((CHUNK, D), jnp.float32),
            pltpu.VMEM((CHUNK, D), jnp.float32),
            pltpu.VMEM((CHUNK,), jnp.int32),
            pltpu.VMEM((CHUNK,), jnp.int32),
            pltpu.VMEM((CHUNK,), jnp.int32),
            pltpu.VMEM((CHUNK,), jnp.int32),
            pltpu.VMEM((CHUNK,), jnp.int32),
            pltpu.VMEM((CHUNK,), jnp.int32),
            pltpu.VMEM((CHUNK,), jnp.int32),
            pltpu.VMEM((CHUNK,), jnp.int32),
            pltpu.VMEM((CHUNK,), jnp.int32),
            pltpu.VMEM_SHARED((N, D), jnp.float32),
            pltpu.SemaphoreType.DMA,
            pltpu.SemaphoreType.DMA,
            pltpu.SemaphoreType.DMA,
            pltpu.SemaphoreType.DMA,
            pltpu.SemaphoreType.DMA,
            pltpu.SemaphoreType.DMA,
            pltpu.SemaphoreType.DMA,
            pltpu.SemaphoreType.DMA,
            pltpu.SemaphoreType.DMA,
            pltpu.SemaphoreType.DMA,
            pltpu.SemaphoreType.DMA,
            pltpu.SemaphoreType.DMA,
        ),
    )
    def k(h_hbm, edata_hbm, out_hbm,
          mg0, mg1, mg2, rb0, rb1, rb2, cb0, cb1, cb2, vb0, vb1, vb2, acc,
          rsem0, rsem1, rsem2, csem0, csem1, csem2,
          vsem0, vsem1, vsem2, gsem0, gsem1, gsem2):
        mg = (mg0, mg1, mg2)
        rb = (rb0, rb1, rb2)
        cb = (cb0, cb1, cb2)
        vb = (vb0, vb1, vb2)
        rsem = (rsem0, rsem1, rsem2)
        csem = (csem0, csem1, csem2)
        vsem = (vsem0, vsem1, vsem2)
        gsem = (gsem0, gsem1, gsem2)
        cid = lax.axis_index("c")
        sid = lax.axis_index("s")
        w = sid * NC + cid

        def start_edata(t, m):
            base = pl.multiple_of(t * 3 * CHUNK, 8)
            pltpu.async_copy(
                edata_hbm.at[w, pl.ds(base, CHUNK)], rb[m], rsem[m])
            pltpu.async_copy(
                edata_hbm.at[w, pl.ds(base + CHUNK, CHUNK)], cb[m], csem[m])
            pltpu.async_copy(
                edata_hbm.at[w, pl.ds(base + 2 * CHUNK, CHUNK)],
                vb[m], vsem[m])

        def wait_edata(m):
            pltpu.make_async_copy(
                edata_hbm.at[w, pl.ds(0, CHUNK)], rb[m], rsem[m]).wait()
            pltpu.make_async_copy(
                edata_hbm.at[w, pl.ds(0, CHUNK)], cb[m], csem[m]).wait()
            pltpu.make_async_copy(
                edata_hbm.at[w, pl.ds(0, CHUNK)], vb[m], vsem[m]).wait()

        def start_gather(m):
            pltpu.async_copy(h_hbm.at[cb[m]], mg[m], gsem[m])

        def wait_gather(m):
            pltpu.make_async_copy(h_hbm.at[cb[m]], mg[m], gsem[m]).wait()

        def scale(m):
            msgs = mg[m]
            vals = vb[m]

            @plsc.parallel_loop(0, CHUNK, unroll=4)
            def _scale(e):
                v = plsc.bitcast(
                    plsc.load_gather(
                        vals, [jnp.full((LANES,), e, jnp.int32)]),
                    jnp.float32)
                for j in range(D // LANES):
                    fsl = pl.ds(j * LANES, LANES)
                    msgs[e, fsl] = msgs[e, fsl] * v

        def scatter(m):
            pltpu.sync_copy(mg[m], acc.at[rb[m]], add=True)

        start_edata(0, 0)
        start_edata(1, 1)
        start_edata(2, 2)

        @plsc.parallel_loop(0, RBLK)
        def _zero(r):
            for j in range(D // LANES):
                mg0[r, pl.ds(j * LANES, LANES)] = jnp.zeros(
                    (LANES,), jnp.float32)

        for t in range(RITERS):
            zb = sid + t * NS

            @pl.when(zb < NRBLK)
            def _():
                z0 = pl.multiple_of(zb * RBLK, 8)
                pltpu.sync_copy(
                    mg0.at[pl.ds(0, RBLK)], acc.at[pl.ds(z0, RBLK)])

        wait_edata(0)
        start_gather(0)
        wait_edata(1)
        start_gather(1)
        plsc.subcore_barrier()

        def slot(i, m, tail=0):
            wait_gather(m)
            scale(m)
            scatter(m)
            if tail < 2:
                g = (m + 2) % 3
                if tail < 1:
                    start_edata(i + 3, m)
                wait_edata(g)
                start_gather(g)

        def triple_body(k3, carry):
            i = k3 * 3
            slot(i, 0)
            slot(i + 1, 1)
            slot(i + 2, 2)
            return carry

        lax.fori_loop(0, (ITERS - 3) // 3, triple_body, 0)

        slot(ITERS - 3, 0, tail=1)
        slot(ITERS - 2, 1, tail=2)
        slot(ITERS - 1, 2, tail=2)
        plsc.subcore_barrier()

        for t in range(RITERS):
            zb = sid + t * NS

            @pl.when(zb < NRBLK)
            def _():
                z0 = pl.multiple_of(zb * RBLK, 8)
                pltpu.sync_copy(
                    acc.at[pl.ds(z0, RBLK)],
                    out_hbm.at[cid, pl.ds(z0, RBLK)],
                )

    return k(h, edata)


def kernel(x, adj_indices, adj_values, W, b):
    h = _matmul_bias(x, W, b)
    pad = E_PAD - E
    rows = jnp.pad(adj_indices[0], (0, pad))
    cols = jnp.pad(adj_indices[1], (0, pad))
    vals = jnp.pad(adj_values, (0, pad))
    edata = jnp.stack(
        [rows.reshape(NW * ITERS, CHUNK),
         cols.reshape(NW * ITERS, CHUNK),
         lax.bitcast_convert_type(vals, jnp.int32).reshape(NW * ITERS, CHUNK)],
        axis=1).reshape(NW, EW)
    parts = _sc_scatter(h, edata)
    return _add2(parts[0], parts[1])
